# baseline (speedup 1.0000x reference)
import jax
import jax.numpy as jnp
import numpy as np
from jax import lax
from jax.experimental import pallas as pl
from jax.experimental.pallas import tpu as pltpu
from jax.experimental.pallas import tpu_sc as plsc

_KB = 8192
_EW = 256
_TB = 4096
_CB = 1024
_NJ = _KB // _CB
_SUB = 1024


def _argmin_kernel(x_ref, kbf2_ref, k2_ref, xl_ref, acc_ref,
                   x2_ref, bestd_ref, besti_ref):
    i = pl.program_id(0)
    j = pl.program_id(1)
    nj = pl.num_programs(1)

    xt = x_ref[0]

    @pl.when(jnp.logical_and(i == 0, j == 0))
    def _init_acc():
        acc_ref[...] = jnp.zeros_like(acc_ref)

    @pl.when(j == 0)
    def _per_token_block():
        x2_ref[...] = jnp.sum(xt * xt, axis=0, keepdims=True)
        acc_ref[0:1, :] += jnp.sum(xt, axis=0, keepdims=True)
        acc_ref[1:2, :] += jnp.sum(xt * xt, axis=0, keepdims=True)

    x_bf = xt.astype(jnp.bfloat16)
    x2 = x2_ref[...]
    bv = None
    for s in range(_CB // _SUB):
        k_bf2 = kbf2_ref[pl.ds(j * _CB + s * _SUB, _SUB), :]
        mms = jax.lax.dot_general(
            k_bf2, x_bf, (((1,), (0,)), ((), ())),
            preferred_element_type=jnp.float32)
        k2s = k2_ref[j, s * _SUB:(s + 1) * _SUB, :]
        ds = (x2 - mms) + jnp.broadcast_to(k2s, (_SUB, _TB))
        av = ds[0:8]
        ai = jnp.zeros((8, _TB), jnp.float32)
        for r in range(1, _SUB // 8):
            dr = ds[8 * r:8 * (r + 1)]
            m = dr < av
            av = jnp.minimum(av, dr)
            ai = jnp.where(m, jnp.float32(r), ai)
        s8 = jax.lax.broadcasted_iota(
            jnp.int32, (8, _TB), 0).astype(jnp.float32)
        gi = (ai * 8.0 + s8) + jnp.float32(s * _SUB)
        if bv is None:
            bv, bi = av, gi
        else:
            m = av < bv
            bi = jnp.where(m, gi, bi)
            bv = jnp.minimum(bv, av)
    bi = bi + jnp.float32(j * _CB)

    first = j == 0
    pv = bestd_ref[...]
    pi = besti_ref[...]
    upd = jnp.logical_or(first, bv < pv)
    nbv = jnp.where(upd, bv, pv)
    nbi = jnp.where(upd, bi, pi)
    bestd_ref[...] = nbv
    besti_ref[...] = nbi

    @pl.when(j == nj - 1)
    def _finish_token_block():
        v, idx = nbv, nbi
        for half in (4, 2, 1):
            v1, v2 = v[0:half], v[half:2 * half]
            i1, i2 = idx[0:half], idx[half:2 * half]
            m = jnp.logical_or(v2 < v1,
                               jnp.logical_and(v2 == v1, i2 < i1))
            v = jnp.where(m, v2, v1)
            idx = jnp.where(m, i2, i1)
        xl_ref[...] = idx.astype(jnp.int32).reshape(1, 1, _TB)
        acc_ref[2:3, :] += v


_SC_CHUNK = 256


def _make_gather(n_tokens):
    info = plsc.get_sparse_core_info()
    nc, ns = info.num_cores, info.num_subcores
    nw = nc * ns
    per_w = n_tokens // nw
    mesh = plsc.VectorSubcoreMesh(core_axis_name="c", subcore_axis_name="s")

    def body(table_hbm, idx_hbm, out_hbm, idx_v, rows_v, sem):
        wid = lax.axis_index("s") * nc + lax.axis_index("c")
        for c in range(per_w // _SC_CHUNK):
            off = wid * per_w + c * _SC_CHUNK
            pltpu.sync_copy(idx_hbm.at[pl.ds(off, _SC_CHUNK)], idx_v)
            pltpu.async_copy(table_hbm.at[idx_v], rows_v, sem).wait()
            pltpu.sync_copy(rows_v, out_hbm.at[pl.ds(off, _SC_CHUNK)])

    return pl.kernel(
        body, mesh=mesh,
        out_type=jax.ShapeDtypeStruct((n_tokens, _EW), jnp.float32),
        scratch_types=[
            pltpu.VMEM((_SC_CHUNK,), jnp.int32),
            pltpu.VMEM((_SC_CHUNK, _EW), jnp.float32),
            pltpu.SemaphoreType.DMA,
        ],
    )


_XB = 1024


def _relayout_kernel(rows_ref, out_ref):
    out_ref[0] = rows_ref[...].T


def kernel(x, k, update_k):
    del update_k
    N, W, T = x.shape
    M = N * T
    ni = M // _TB
    tpn = T // _TB
    k_bf2 = (2.0 * k).astype(jnp.bfloat16)
    k2 = jnp.sum(k.T ** 2, axis=0)
    k2b = k2.reshape(_NJ, _CB, 1)

    xl3, acc = pl.pallas_call(
        _argmin_kernel,
        grid=(ni, _NJ),
        in_specs=[
            pl.BlockSpec((1, W, _TB), lambda i, j: (i // tpn, 0, i % tpn)),
            pl.BlockSpec((_KB, _EW), lambda i, j: (0, 0)),
            pl.BlockSpec((_NJ, _CB, 1), lambda i, j: (0, 0, 0)),
        ],
        out_specs=[
            pl.BlockSpec((1, 1, _TB), lambda i, j: (i, 0, 0)),
            pl.BlockSpec((8, _TB), lambda i, j: (0, 0)),
        ],
        out_shape=[
            jax.ShapeDtypeStruct((ni, 1, _TB), jnp.int32),
            jax.ShapeDtypeStruct((8, _TB), jnp.float32),
        ],
        scratch_shapes=[
            pltpu.VMEM((1, _TB), jnp.float32),
            pltpu.VMEM((8, _TB), jnp.float32),
            pltpu.VMEM((8, _TB), jnp.float32),
        ],
    )(x, k_bf2, k2b)

    x_l = xl3.reshape(N, T)
    rows = _make_gather(M)(k, xl3.reshape(M))

    xd = pl.pallas_call(
        _relayout_kernel,
        grid=(M // _XB,),
        in_specs=[pl.BlockSpec((_XB, _EW), lambda i: (i, 0))],
        out_specs=pl.BlockSpec(
            (1, W, _XB), lambda i: (i // (T // _XB), 0, i % (T // _XB))),
        out_shape=jax.ShapeDtypeStruct((N, W, T), jnp.float32),
    )(rows)

    s1 = jnp.sum(acc[0])
    s2 = jnp.sum(acc[1])
    fitsum = jnp.sum(acc[2])
    ne = float(M * W)
    fit = fitsum / float(M)
    commit_loss = fitsum / ne
    prenorm = jnp.sqrt(jnp.maximum(s2 - s1 * s1 / ne, 0.0)) / np.sqrt(ne)
    return (x_l, xd, commit_loss, fit, prenorm)


# speedup vs baseline: 1.0944x; 1.0944x over previous
import jax
import jax.numpy as jnp
import numpy as np
from jax import lax
from jax.experimental import pallas as pl
from jax.experimental.pallas import tpu as pltpu
from jax.experimental.pallas import tpu_sc as plsc

_KB = 8192
_EW = 256
_TB = 4096
_CB = 2048
_NJ = _KB // _CB
_SUB = 512


def _argmin_kernel(x_ref, kbf2_ref, k2_ref, xl_ref, acc_ref,
                   x2_ref, bestd_ref, besti_ref):
    i = pl.program_id(0)
    j = pl.program_id(1)
    nj = pl.num_programs(1)

    xt = x_ref[0]

    @pl.when(jnp.logical_and(i == 0, j == 0))
    def _init_acc():
        acc_ref[...] = jnp.zeros_like(acc_ref)

    @pl.when(j == 0)
    def _per_token_block():
        x2_ref[...] = jnp.sum(xt * xt, axis=0, keepdims=True)
        acc_ref[0:1, :] += jnp.sum(xt, axis=0, keepdims=True)
        acc_ref[1:2, :] += jnp.sum(xt * xt, axis=0, keepdims=True)

    x_bf = xt.astype(jnp.bfloat16)
    x2 = x2_ref[...]
    bv = None
    for s in range(_CB // _SUB):
        k_bf2 = kbf2_ref[pl.ds(j * _CB + s * _SUB, _SUB), :]
        mms = jax.lax.dot_general(
            k_bf2, x_bf, (((1,), (0,)), ((), ())),
            preferred_element_type=jnp.float32)
        k2s = k2_ref[j, s * _SUB:(s + 1) * _SUB, :]
        ds = (x2 - mms) + jnp.broadcast_to(k2s, (_SUB, _TB))
        av = ds[0:8]
        ai = jnp.zeros((8, _TB), jnp.float32)
        for r in range(1, _SUB // 8):
            dr = ds[8 * r:8 * (r + 1)]
            m = dr < av
            av = jnp.minimum(av, dr)
            ai = jnp.where(m, jnp.float32(r), ai)
        s8 = jax.lax.broadcasted_iota(
            jnp.int32, (8, _TB), 0).astype(jnp.float32)
        gi = (ai * 8.0 + s8) + jnp.float32(s * _SUB)
        if bv is None:
            bv, bi = av, gi
        else:
            m = av < bv
            bi = jnp.where(m, gi, bi)
            bv = jnp.minimum(bv, av)
    bi = bi + jnp.float32(j * _CB)

    first = j == 0
    pv = bestd_ref[...]
    pi = besti_ref[...]
    upd = jnp.logical_or(first, bv < pv)
    nbv = jnp.where(upd, bv, pv)
    nbi = jnp.where(upd, bi, pi)
    bestd_ref[...] = nbv
    besti_ref[...] = nbi

    @pl.when(j == nj - 1)
    def _finish_token_block():
        v, idx = nbv, nbi
        for half in (4, 2, 1):
            v1, v2 = v[0:half], v[half:2 * half]
            i1, i2 = idx[0:half], idx[half:2 * half]
            m = jnp.logical_or(v2 < v1,
                               jnp.logical_and(v2 == v1, i2 < i1))
            v = jnp.where(m, v2, v1)
            idx = jnp.where(m, i2, i1)
        xl_ref[...] = idx.astype(jnp.int32).reshape(1, 1, _TB)
        acc_ref[2:3, :] += v


_SC_CHUNK = 256


def _make_gather(n_tokens):
    info = plsc.get_sparse_core_info()
    nc, ns = info.num_cores, info.num_subcores
    nw = nc * ns
    per_w = n_tokens // nw
    mesh = plsc.VectorSubcoreMesh(core_axis_name="c", subcore_axis_name="s")

    def body(table_hbm, idx_hbm, out_hbm, idx_v, rows_v, sem):
        wid = lax.axis_index("s") * nc + lax.axis_index("c")
        for c in range(per_w // _SC_CHUNK):
            off = wid * per_w + c * _SC_CHUNK
            pltpu.sync_copy(idx_hbm.at[pl.ds(off, _SC_CHUNK)], idx_v)
            pltpu.async_copy(table_hbm.at[idx_v], rows_v, sem).wait()
            pltpu.sync_copy(rows_v, out_hbm.at[pl.ds(off, _SC_CHUNK)])

    return pl.kernel(
        body, mesh=mesh,
        out_type=jax.ShapeDtypeStruct((n_tokens, _EW), jnp.float32),
        scratch_types=[
            pltpu.VMEM((_SC_CHUNK,), jnp.int32),
            pltpu.VMEM((_SC_CHUNK, _EW), jnp.float32),
            pltpu.SemaphoreType.DMA,
        ],
    )


_XB = 1024


def _relayout_kernel(rows_ref, out_ref):
    out_ref[0] = rows_ref[...].T


def kernel(x, k, update_k):
    del update_k
    N, W, T = x.shape
    M = N * T
    ni = M // _TB
    tpn = T // _TB
    k_bf2 = (2.0 * k).astype(jnp.bfloat16)
    k2 = jnp.sum(k.T ** 2, axis=0)
    k2b = k2.reshape(_NJ, _CB, 1)

    xl3, acc = pl.pallas_call(
        _argmin_kernel,
        grid=(ni, _NJ),
        in_specs=[
            pl.BlockSpec((1, W, _TB), lambda i, j: (i // tpn, 0, i % tpn)),
            pl.BlockSpec((_KB, _EW), lambda i, j: (0, 0)),
            pl.BlockSpec((_NJ, _CB, 1), lambda i, j: (0, 0, 0)),
        ],
        out_specs=[
            pl.BlockSpec((1, 1, _TB), lambda i, j: (i, 0, 0)),
            pl.BlockSpec((8, _TB), lambda i, j: (0, 0)),
        ],
        out_shape=[
            jax.ShapeDtypeStruct((ni, 1, _TB), jnp.int32),
            jax.ShapeDtypeStruct((8, _TB), jnp.float32),
        ],
        scratch_shapes=[
            pltpu.VMEM((1, _TB), jnp.float32),
            pltpu.VMEM((8, _TB), jnp.float32),
            pltpu.VMEM((8, _TB), jnp.float32),
        ],
    )(x, k_bf2, k2b)

    x_l = xl3.reshape(N, T)
    rows = _make_gather(M)(k, xl3.reshape(M))

    xd = pl.pallas_call(
        _relayout_kernel,
        grid=(M // _XB,),
        in_specs=[pl.BlockSpec((_XB, _EW), lambda i: (i, 0))],
        out_specs=pl.BlockSpec(
            (1, W, _XB), lambda i: (i // (T // _XB), 0, i % (T // _XB))),
        out_shape=jax.ShapeDtypeStruct((N, W, T), jnp.float32),
    )(rows)

    s1 = jnp.sum(acc[0])
    s2 = jnp.sum(acc[1])
    fitsum = jnp.sum(acc[2])
    ne = float(M * W)
    fit = fitsum / float(M)
    commit_loss = fitsum / ne
    prenorm = jnp.sqrt(jnp.maximum(s2 - s1 * s1 / ne, 0.0)) / np.sqrt(ne)
    return (x_l, xd, commit_loss, fit, prenorm)


# speedup vs baseline: 1.1333x; 1.0355x over previous
import jax
import jax.numpy as jnp
import numpy as np
from jax import lax
from jax.experimental import pallas as pl
from jax.experimental.pallas import tpu as pltpu
from jax.experimental.pallas import tpu_sc as plsc

_KB = 8192
_EW = 256
_TB = 4096
_CB = 4096
_NJ = _KB // _CB
_SUB = 512


def _argmin_kernel(x_ref, kbf2_ref, k2_ref, xl_ref, acc_ref,
                   x2_ref, bestd_ref, besti_ref):
    i = pl.program_id(0)
    j = pl.program_id(1)
    nj = pl.num_programs(1)

    xt = x_ref[0]

    @pl.when(jnp.logical_and(i == 0, j == 0))
    def _init_acc():
        acc_ref[...] = jnp.zeros_like(acc_ref)

    @pl.when(j == 0)
    def _per_token_block():
        x2_ref[...] = jnp.sum(xt * xt, axis=0, keepdims=True)
        acc_ref[0:1, :] += jnp.sum(xt, axis=0, keepdims=True)
        acc_ref[1:2, :] += jnp.sum(xt * xt, axis=0, keepdims=True)

    x_bf = xt.astype(jnp.bfloat16)
    x2 = x2_ref[...]
    bv = None
    for s in range(_CB // _SUB):
        k_bf2 = kbf2_ref[pl.ds(j * _CB + s * _SUB, _SUB), :]
        mms = jax.lax.dot_general(
            k_bf2, x_bf, (((1,), (0,)), ((), ())),
            preferred_element_type=jnp.float32)
        k2s = k2_ref[j, s * _SUB:(s + 1) * _SUB, :]
        ds = (x2 - mms) + jnp.broadcast_to(k2s, (_SUB, _TB))
        av = ds[0:8]
        ai = jnp.zeros((8, _TB), jnp.float32)
        for r in range(1, _SUB // 8):
            dr = ds[8 * r:8 * (r + 1)]
            m = dr < av
            av = jnp.minimum(av, dr)
            ai = jnp.where(m, jnp.float32(r), ai)
        s8 = jax.lax.broadcasted_iota(
            jnp.int32, (8, _TB), 0).astype(jnp.float32)
        gi = (ai * 8.0 + s8) + jnp.float32(s * _SUB)
        if bv is None:
            bv, bi = av, gi
        else:
            m = av < bv
            bi = jnp.where(m, gi, bi)
            bv = jnp.minimum(bv, av)
    bi = bi + jnp.float32(j * _CB)

    first = j == 0
    pv = bestd_ref[...]
    pi = besti_ref[...]
    upd = jnp.logical_or(first, bv < pv)
    nbv = jnp.where(upd, bv, pv)
    nbi = jnp.where(upd, bi, pi)
    bestd_ref[...] = nbv
    besti_ref[...] = nbi

    @pl.when(j == nj - 1)
    def _finish_token_block():
        v, idx = nbv, nbi
        for half in (4, 2, 1):
            v1, v2 = v[0:half], v[half:2 * half]
            i1, i2 = idx[0:half], idx[half:2 * half]
            m = jnp.logical_or(v2 < v1,
                               jnp.logical_and(v2 == v1, i2 < i1))
            v = jnp.where(m, v2, v1)
            idx = jnp.where(m, i2, i1)
        xl_ref[...] = idx.astype(jnp.int32).reshape(1, 1, _TB)
        acc_ref[2:3, :] += v


_SC_CHUNK = 256


def _make_gather(n_tokens):
    info = plsc.get_sparse_core_info()
    nc, ns = info.num_cores, info.num_subcores
    nw = nc * ns
    per_w = n_tokens // nw
    mesh = plsc.VectorSubcoreMesh(core_axis_name="c", subcore_axis_name="s")

    def body(table_hbm, idx_hbm, out_hbm, idx_v, rows_v, sem):
        wid = lax.axis_index("s") * nc + lax.axis_index("c")
        for c in range(per_w // _SC_CHUNK):
            off = wid * per_w + c * _SC_CHUNK
            pltpu.sync_copy(idx_hbm.at[pl.ds(off, _SC_CHUNK)], idx_v)
            pltpu.async_copy(table_hbm.at[idx_v], rows_v, sem).wait()
            pltpu.sync_copy(rows_v, out_hbm.at[pl.ds(off, _SC_CHUNK)])

    return pl.kernel(
        body, mesh=mesh,
        out_type=jax.ShapeDtypeStruct((n_tokens, _EW), jnp.float32),
        scratch_types=[
            pltpu.VMEM((_SC_CHUNK,), jnp.int32),
            pltpu.VMEM((_SC_CHUNK, _EW), jnp.float32),
            pltpu.SemaphoreType.DMA,
        ],
    )


_XB = 1024


def _relayout_kernel(rows_ref, out_ref):
    out_ref[0] = rows_ref[...].T


def kernel(x, k, update_k):
    del update_k
    N, W, T = x.shape
    M = N * T
    ni = M // _TB
    tpn = T // _TB
    k_bf2 = (2.0 * k).astype(jnp.bfloat16)
    k2 = jnp.sum(k.T ** 2, axis=0)
    k2b = k2.reshape(_NJ, _CB, 1)

    xl3, acc = pl.pallas_call(
        _argmin_kernel,
        grid=(ni, _NJ),
        in_specs=[
            pl.BlockSpec((1, W, _TB), lambda i, j: (i // tpn, 0, i % tpn)),
            pl.BlockSpec((_KB, _EW), lambda i, j: (0, 0)),
            pl.BlockSpec((_NJ, _CB, 1), lambda i, j: (0, 0, 0)),
        ],
        out_specs=[
            pl.BlockSpec((1, 1, _TB), lambda i, j: (i, 0, 0)),
            pl.BlockSpec((8, _TB), lambda i, j: (0, 0)),
        ],
        out_shape=[
            jax.ShapeDtypeStruct((ni, 1, _TB), jnp.int32),
            jax.ShapeDtypeStruct((8, _TB), jnp.float32),
        ],
        scratch_shapes=[
            pltpu.VMEM((1, _TB), jnp.float32),
            pltpu.VMEM((8, _TB), jnp.float32),
            pltpu.VMEM((8, _TB), jnp.float32),
        ],
    )(x, k_bf2, k2b)

    x_l = xl3.reshape(N, T)
    rows = _make_gather(M)(k, xl3.reshape(M))

    xd = pl.pallas_call(
        _relayout_kernel,
        grid=(M // _XB,),
        in_specs=[pl.BlockSpec((_XB, _EW), lambda i: (i, 0))],
        out_specs=pl.BlockSpec(
            (1, W, _XB), lambda i: (i // (T // _XB), 0, i % (T // _XB))),
        out_shape=jax.ShapeDtypeStruct((N, W, T), jnp.float32),
    )(rows)

    s1 = jnp.sum(acc[0])
    s2 = jnp.sum(acc[1])
    fitsum = jnp.sum(acc[2])
    ne = float(M * W)
    fit = fitsum / float(M)
    commit_loss = fitsum / ne
    prenorm = jnp.sqrt(jnp.maximum(s2 - s1 * s1 / ne, 0.0)) / np.sqrt(ne)
    return (x_l, xd, commit_loss, fit, prenorm)


# speedup vs baseline: 1.1584x; 1.0222x over previous
import jax
import jax.numpy as jnp
import numpy as np
from jax import lax
from jax.experimental import pallas as pl
from jax.experimental.pallas import tpu as pltpu
from jax.experimental.pallas import tpu_sc as plsc

_KB = 8192
_EW = 256
_TB = 4096
_CB = 8192
_NJ = _KB // _CB
_SUB = 512


def _argmin_kernel(x_ref, kbf2_ref, k2_ref, xl_ref, acc_ref,
                   x2_ref, bestd_ref, besti_ref):
    i = pl.program_id(0)
    j = pl.program_id(1)
    nj = pl.num_programs(1)

    xt = x_ref[0]

    @pl.when(jnp.logical_and(i == 0, j == 0))
    def _init_acc():
        acc_ref[...] = jnp.zeros_like(acc_ref)

    @pl.when(j == 0)
    def _per_token_block():
        x2_ref[...] = jnp.sum(xt * xt, axis=0, keepdims=True)
        acc_ref[0:1, :] += jnp.sum(xt, axis=0, keepdims=True)
        acc_ref[1:2, :] += jnp.sum(xt * xt, axis=0, keepdims=True)

    x_bf = xt.astype(jnp.bfloat16)
    x2 = x2_ref[...]
    bv = None
    for s in range(_CB // _SUB):
        k_bf2 = kbf2_ref[pl.ds(j * _CB + s * _SUB, _SUB), :]
        mms = jax.lax.dot_general(
            k_bf2, x_bf, (((1,), (0,)), ((), ())),
            preferred_element_type=jnp.float32)
        k2s = k2_ref[j, s * _SUB:(s + 1) * _SUB, :]
        ds = (x2 - mms) + jnp.broadcast_to(k2s, (_SUB, _TB))
        av = ds[0:8]
        ai = jnp.zeros((8, _TB), jnp.float32)
        for r in range(1, _SUB // 8):
            dr = ds[8 * r:8 * (r + 1)]
            m = dr < av
            av = jnp.minimum(av, dr)
            ai = jnp.where(m, jnp.float32(r), ai)
        s8 = jax.lax.broadcasted_iota(
            jnp.int32, (8, _TB), 0).astype(jnp.float32)
        gi = (ai * 8.0 + s8) + jnp.float32(s * _SUB)
        if bv is None:
            bv, bi = av, gi
        else:
            m = av < bv
            bi = jnp.where(m, gi, bi)
            bv = jnp.minimum(bv, av)
    bi = bi + jnp.float32(j * _CB)

    first = j == 0
    pv = bestd_ref[...]
    pi = besti_ref[...]
    upd = jnp.logical_or(first, bv < pv)
    nbv = jnp.where(upd, bv, pv)
    nbi = jnp.where(upd, bi, pi)
    bestd_ref[...] = nbv
    besti_ref[...] = nbi

    @pl.when(j == nj - 1)
    def _finish_token_block():
        v, idx = nbv, nbi
        for half in (4, 2, 1):
            v1, v2 = v[0:half], v[half:2 * half]
            i1, i2 = idx[0:half], idx[half:2 * half]
            m = jnp.logical_or(v2 < v1,
                               jnp.logical_and(v2 == v1, i2 < i1))
            v = jnp.where(m, v2, v1)
            idx = jnp.where(m, i2, i1)
        xl_ref[...] = idx.astype(jnp.int32).reshape(1, 1, _TB)
        acc_ref[2:3, :] += v


_SC_CHUNK = 256


def _make_gather(n_tokens):
    info = plsc.get_sparse_core_info()
    nc, ns = info.num_cores, info.num_subcores
    nw = nc * ns
    per_w = n_tokens // nw
    mesh = plsc.VectorSubcoreMesh(core_axis_name="c", subcore_axis_name="s")

    def body(table_hbm, idx_hbm, out_hbm, idx_v, rows_v, sem):
        wid = lax.axis_index("s") * nc + lax.axis_index("c")
        for c in range(per_w // _SC_CHUNK):
            off = wid * per_w + c * _SC_CHUNK
            pltpu.sync_copy(idx_hbm.at[pl.ds(off, _SC_CHUNK)], idx_v)
            pltpu.async_copy(table_hbm.at[idx_v], rows_v, sem).wait()
            pltpu.sync_copy(rows_v, out_hbm.at[pl.ds(off, _SC_CHUNK)])

    return pl.kernel(
        body, mesh=mesh,
        out_type=jax.ShapeDtypeStruct((n_tokens, _EW), jnp.float32),
        scratch_types=[
            pltpu.VMEM((_SC_CHUNK,), jnp.int32),
            pltpu.VMEM((_SC_CHUNK, _EW), jnp.float32),
            pltpu.SemaphoreType.DMA,
        ],
    )


_XB = 1024


def _relayout_kernel(rows_ref, out_ref):
    out_ref[0] = rows_ref[...].T


def kernel(x, k, update_k):
    del update_k
    N, W, T = x.shape
    M = N * T
    ni = M // _TB
    tpn = T // _TB
    k_bf2 = (2.0 * k).astype(jnp.bfloat16)
    k2 = jnp.sum(k.T ** 2, axis=0)
    k2b = k2.reshape(_NJ, _CB, 1)

    xl3, acc = pl.pallas_call(
        _argmin_kernel,
        grid=(ni, _NJ),
        in_specs=[
            pl.BlockSpec((1, W, _TB), lambda i, j: (i // tpn, 0, i % tpn)),
            pl.BlockSpec((_KB, _EW), lambda i, j: (0, 0)),
            pl.BlockSpec((_NJ, _CB, 1), lambda i, j: (0, 0, 0)),
        ],
        out_specs=[
            pl.BlockSpec((1, 1, _TB), lambda i, j: (i, 0, 0)),
            pl.BlockSpec((8, _TB), lambda i, j: (0, 0)),
        ],
        out_shape=[
            jax.ShapeDtypeStruct((ni, 1, _TB), jnp.int32),
            jax.ShapeDtypeStruct((8, _TB), jnp.float32),
        ],
        scratch_shapes=[
            pltpu.VMEM((1, _TB), jnp.float32),
            pltpu.VMEM((8, _TB), jnp.float32),
            pltpu.VMEM((8, _TB), jnp.float32),
        ],
    )(x, k_bf2, k2b)

    x_l = xl3.reshape(N, T)
    rows = _make_gather(M)(k, xl3.reshape(M))

    xd = pl.pallas_call(
        _relayout_kernel,
        grid=(M // _XB,),
        in_specs=[pl.BlockSpec((_XB, _EW), lambda i: (i, 0))],
        out_specs=pl.BlockSpec(
            (1, W, _XB), lambda i: (i // (T // _XB), 0, i % (T // _XB))),
        out_shape=jax.ShapeDtypeStruct((N, W, T), jnp.float32),
    )(rows)

    s1 = jnp.sum(acc[0])
    s2 = jnp.sum(acc[1])
    fitsum = jnp.sum(acc[2])
    ne = float(M * W)
    fit = fitsum / float(M)
    commit_loss = fitsum / ne
    prenorm = jnp.sqrt(jnp.maximum(s2 - s1 * s1 / ne, 0.0)) / np.sqrt(ne)
    return (x_l, xd, commit_loss, fit, prenorm)
